# baseline (speedup 1.0000x reference)
import jax
import jax.numpy as jnp
from jax.experimental import pallas as pl
from jax.experimental.pallas import tpu as pltpu

H = 128
BF = jnp.bfloat16
F32 = jnp.float32


def _bdot(x, y):
    return jnp.dot(x.astype(BF), y.astype(BF), preferred_element_type=F32)


def _obj_chain(feat, osupp_ref, obj0, wv_ref, ws_ref, wo_ref, bo_ref,
               w_obj_ref):
    supp_term = (osupp_ref[...].astype(BF).astype(F32)
                 * ws_ref[...].astype(BF).astype(F32))
    attn_o = (_bdot(feat, wv_ref[...]) + supp_term
              + _bdot(obj0, wo_ref[...]) + bo_ref[...])
    agg = jax.lax.dot_general(attn_o.astype(BF), feat.astype(BF),
                              (((0,), (0,)), ((), ())),
                              preferred_element_type=F32)
    w_obj = w_obj_ref[...]
    return jax.nn.relu(_bdot(obj0, w_obj[:H]) + _bdot(agg, w_obj[H:]))


def _a_coef(wa_c_ref, feat):
    return jax.lax.dot_general(
        wa_c_ref[...].astype(BF), feat.astype(BF), (((0,), (1,)), ((), ())),
        preferred_element_type=F32)


def _strip(s_l_ref, s_r_ref, feat, dest, a_t, wr_c_ref, sw_ref, b_ref,
           obj, wa_ref, wb_ref):
    nh = s_l_ref.shape[1]
    r = _bdot(dest, wr_c_ref[...])
    sw = sw_ref[...]
    b = b_ref[...]
    halves = []
    for s_ref, lo in ((s_l_ref, 0), (s_r_ref, nh)):
        s = s_ref[...]
        attn = ((a_t[:, lo:lo + nh] + s * sw) + r) + b
        halves.append(jnp.where(s != 0.0, attn, 0.0))
    acc = _bdot(jnp.concatenate(halves, axis=1), feat)
    wa = wa_ref[...]
    wb = wb_ref[...]
    oc = jax.nn.relu(_bdot(obj, wa[:H]) + _bdot(dest, wa[H:]))
    return jax.nn.relu(_bdot(oc, wb[:H]) + _bdot(acc, wb[H:]))


def _layer_body(cv_l_ref, cv_r_ref, vc_l_ref, vc_r_ref, colh_ref, rowh_ref,
                vo_supp_ref, co_supp_ref, obj0_ref,
                wv0_ref, ws0_ref, wo0_ref, bo0_ref, vo_w_ref,
                wa0_ref, wr0_ref, sw0_ref, b0_ref, oc_w_ref, vc_w_ref,
                wv1_ref, ws1_ref, wo1_ref, bo1_ref, co_w_ref,
                wa1_ref, wr1_ref, sw1_ref, b1_ref, ov_w_ref, cv_w_ref,
                row_out_ref, col_out_ref, cv_out_ref, vc_out_ref, obj_out_ref,
                rown_scr, obj1_scr, obj2_scr, a0_scr, a1_scr, sem_l, sem_r):
    p = pl.program_id(0)
    i = pl.program_id(1)
    bi = cv_l_ref.shape[0]
    nh = cv_l_ref.shape[1]
    colh = colh_ref[...]

    @pl.when(p == 0)
    def _():
        cps = [pltpu.make_async_copy(
                   cv_l_ref, cv_out_ref.at[pl.ds(i * bi, bi), pl.ds(0, nh)],
                   sem_l),
               pltpu.make_async_copy(
                   cv_r_ref, cv_out_ref.at[pl.ds(i * bi, bi), pl.ds(nh, nh)],
                   sem_r)]
        for cp in cps:
            cp.start()

        @pl.when(i == 0)
        def _():
            obj1_scr[...] = _obj_chain(colh, vo_supp_ref, obj0_ref[...],
                                       wv0_ref, ws0_ref, wo0_ref, bo0_ref,
                                       vo_w_ref)
            a0_scr[...] = _a_coef(wa0_ref, colh)

        dest = rowh_ref[pl.ds(i * bi, bi), :]
        row_next = _strip(cv_l_ref, cv_r_ref, colh, dest, a0_scr[...],
                          wr0_ref, sw0_ref, b0_ref, obj1_scr[...],
                          oc_w_ref, vc_w_ref)
        row_out_ref[...] = row_next
        rown_scr[pl.ds(i * bi, bi), :] = row_next
        for cp in cps:
            cp.wait()

    @pl.when(p == 1)
    def _():
        cps = [pltpu.make_async_copy(
                   vc_l_ref, vc_out_ref.at[pl.ds(i * bi, bi), pl.ds(0, nh)],
                   sem_l),
               pltpu.make_async_copy(
                   vc_r_ref, vc_out_ref.at[pl.ds(i * bi, bi), pl.ds(nh, nh)],
                   sem_r)]
        for cp in cps:
            cp.start()
        rown = rown_scr[...]

        @pl.when(i == 0)
        def _():
            obj2 = _obj_chain(rown, co_supp_ref, obj1_scr[...],
                              wv1_ref, ws1_ref, wo1_ref, bo1_ref, co_w_ref)
            obj2_scr[...] = obj2
            obj_out_ref[...] = obj2
            a1_scr[...] = _a_coef(wa1_ref, rown)

        dest = colh_ref[pl.ds(i * bi, bi), :]
        col_next = _strip(vc_l_ref, vc_r_ref, rown, dest, a1_scr[...],
                          wr1_ref, sw1_ref, b1_ref, obj2_scr[...],
                          ov_w_ref, cv_w_ref)
        col_out_ref[...] = col_next
        for cp in cps:
            cp.wait()


def kernel(col_hidden, row_hidden, obj_hidden, cv_supp, vc_supp, vo_supp,
           co_supp, vc_w, cv_w, co_w, oc_w, vo_w, ov_w,
           attn_vo_w, attn_vo_b, attn_cv_w, attn_cv_b,
           attn_co_w, attn_co_b, attn_vc_w, attn_vc_b):
    n = col_hidden.shape[0]
    bi = 512
    ni = n // bi

    def full(shape):
        return pl.BlockSpec(shape, lambda p, i: tuple(0 for _ in shape))

    cv_spec = lambda h: pl.BlockSpec(
        (bi, n // 2), lambda p, i: (i * (1 - p) + (ni - 1) * p, h))
    vc_spec = lambda h: pl.BlockSpec(
        (bi, n // 2), lambda p, i: (i * p, h))

    row_next, col_next, cv_copy, vc_copy, obj2 = pl.pallas_call(
        _layer_body,
        grid=(2, ni),
        in_specs=[
            cv_spec(0), cv_spec(1),
            vc_spec(0), vc_spec(1),
            full((n, H)),
            full((n, H)),
            full((n, 1)),
            full((n, 1)),
            full((1, H)),
            full((H, 1)), full((1, 1)), full((H, 1)), full((1, 1)),
            full((2 * H, H)),
            full((H, 1)), full((H, 1)), full((1, 1)), full((1, 1)),
            full((2 * H, H)), full((2 * H, H)),
            full((H, 1)), full((1, 1)), full((H, 1)), full((1, 1)),
            full((2 * H, H)),
            full((H, 1)), full((H, 1)), full((1, 1)), full((1, 1)),
            full((2 * H, H)), full((2 * H, H)),
        ],
        out_specs=(
            pl.BlockSpec((bi, H), lambda p, i: (i * (1 - p) + (ni - 1) * p, 0)),
            pl.BlockSpec((bi, H), lambda p, i: (i * p, 0)),
            pl.BlockSpec(memory_space=pl.ANY),
            pl.BlockSpec(memory_space=pl.ANY),
            full((1, H)),
        ),
        out_shape=(
            jax.ShapeDtypeStruct((n, H), F32),
            jax.ShapeDtypeStruct((n, H), F32),
            jax.ShapeDtypeStruct((n, n), F32),
            jax.ShapeDtypeStruct((n, n), F32),
            jax.ShapeDtypeStruct((1, H), F32),
        ),
        scratch_shapes=[
            pltpu.VMEM((n, H), F32),
            pltpu.VMEM((1, H), F32),
            pltpu.VMEM((1, H), F32),
            pltpu.VMEM((1, n), F32),
            pltpu.VMEM((1, n), F32),
            pltpu.SemaphoreType.DMA,
            pltpu.SemaphoreType.DMA,
        ],
    )(cv_supp[0], cv_supp[0], vc_supp[0], vc_supp[0], col_hidden, row_hidden,
      vo_supp, co_supp, obj_hidden,
      attn_vo_w[:H], attn_vo_w[H:H + 1], attn_vo_w[H + 1:],
      attn_vo_b.reshape(1, 1), vo_w,
      attn_cv_w[:H], attn_cv_w[H + 1:], attn_cv_w[H:H + 1],
      attn_cv_b.reshape(1, 1), oc_w, vc_w,
      attn_co_w[:H], attn_co_w[H:H + 1], attn_co_w[H + 1:],
      attn_co_b.reshape(1, 1), co_w,
      attn_vc_w[:H], attn_vc_w[H + 1:], attn_vc_w[H:H + 1],
      attn_vc_b.reshape(1, 1), ov_w, cv_w)
    return (col_next, row_next, obj2, cv_copy[None], vc_copy[None],
            vo_supp, co_supp)


# speedup vs baseline: 1.0238x; 1.0238x over previous
import jax
import jax.numpy as jnp
from jax.experimental import pallas as pl
from jax.experimental.pallas import tpu as pltpu

H = 128
BF = jnp.bfloat16
F32 = jnp.float32


def _bdot(x, y):
    return jnp.dot(x.astype(BF), y.astype(BF), preferred_element_type=F32)


def _flash_body(s_l_ref, s_r_ref, feat_ref, dest_ref, osupp_ref, obj_ref,
                wv_ref, ws_ref, wo_ref, bo_ref, w_obj_ref, wa_c_ref, wr_c_ref,
                sw_ref, b_ref, wa_ref, wb_ref, out_ref, s_out_ref,
                obj_out_ref, obj_scr, a_t_scr, sem_l, sem_r):
    i = pl.program_id(0)
    feat = feat_ref[...]
    bi = s_l_ref.shape[0]
    nh2 = s_l_ref.shape[1]
    cp_l = pltpu.make_async_copy(
        s_l_ref, s_out_ref.at[pl.ds(i * bi, bi), pl.ds(0, nh2)], sem_l)
    cp_r = pltpu.make_async_copy(
        s_r_ref, s_out_ref.at[pl.ds(i * bi, bi), pl.ds(nh2, nh2)], sem_r)
    cp_l.start()
    cp_r.start()

    @pl.when(i == 0)
    def _():
        obj0 = obj_ref[...]
        supp_term = (osupp_ref[...].astype(BF).astype(F32)
                     * ws_ref[...].astype(BF).astype(F32))
        attn_o = (_bdot(feat, wv_ref[...]) + supp_term
                  + _bdot(obj0, wo_ref[...]) + bo_ref[...])
        agg = jax.lax.dot_general(attn_o.astype(BF), feat.astype(BF),
                                  (((0,), (0,)), ((), ())),
                                  preferred_element_type=F32)
        w_obj = w_obj_ref[...]
        obj_next = jax.nn.relu(_bdot(obj0, w_obj[:H])
                               + _bdot(agg, w_obj[H:]))
        obj_scr[...] = obj_next
        obj_out_ref[...] = obj_next
        a_t_scr[...] = jax.lax.dot_general(
            wa_c_ref[...].astype(BF), feat.astype(BF), (((0,), (1,)), ((), ())),
            preferred_element_type=F32)

    dest = dest_ref[...]
    r = _bdot(dest, wr_c_ref[...])
    nh = feat.shape[0] // 2
    sw = sw_ref[...]
    b = b_ref[...]
    a_t = a_t_scr[...]
    halves = []
    for s_ref, lo in ((s_l_ref, 0), (s_r_ref, nh)):
        s = s_ref[...]
        attn = ((a_t[:, lo:lo + nh] + s * sw) + r) + b
        halves.append(jnp.where(s != 0.0, attn, 0.0))
    acc = _bdot(jnp.concatenate(halves, axis=1), feat)
    wa = wa_ref[...]
    wb = wb_ref[...]
    oc = jax.nn.relu(_bdot(obj_scr[...], wa[:H]) + _bdot(dest, wa[H:]))
    out_ref[...] = jax.nn.relu(_bdot(oc, wb[:H]) + _bdot(acc, wb[H:]))
    cp_l.wait()
    cp_r.wait()


def _flash_conv(s2d, feat, dest, osupp, obj, wv, ws, wo, bo, w_obj,
                wa_c, wr_c, sw, b, wa, wb, bi=512):
    ni_dim, n = s2d.shape
    ni = ni_dim // bi

    def full(shape):
        return pl.BlockSpec(shape, lambda i: tuple(0 for _ in shape))

    return pl.pallas_call(
        _flash_body,
        grid=(ni,),
        in_specs=[
            pl.BlockSpec((bi, n // 2), lambda i: (i, 0)),
            pl.BlockSpec((bi, n // 2), lambda i: (i, 1)),
            full((n, H)),
            pl.BlockSpec((bi, H), lambda i: (i, 0)),
            full((n, 1)),
            full((1, H)),
            full((H, 1)),
            full((1, 1)),
            full((H, 1)),
            full((1, 1)),
            full((2 * H, H)),
            full((H, 1)),
            full((H, 1)),
            full((1, 1)),
            full((1, 1)),
            full((2 * H, H)),
            full((2 * H, H)),
        ],
        out_specs=(
            pl.BlockSpec((bi, H), lambda i: (i, 0)),
            pl.BlockSpec(memory_space=pl.ANY),
            full((1, H)),
        ),
        out_shape=(
            jax.ShapeDtypeStruct((ni_dim, H), F32),
            jax.ShapeDtypeStruct((ni_dim, n), F32),
            jax.ShapeDtypeStruct((1, H), F32),
        ),
        scratch_shapes=[
            pltpu.VMEM((1, H), F32),
            pltpu.VMEM((1, n), F32),
            pltpu.SemaphoreType.DMA,
            pltpu.SemaphoreType.DMA,
        ],
    )(s2d, s2d, feat, dest, osupp, obj, wv, ws, wo, bo, w_obj,
      wa_c, wr_c, sw, b, wa, wb)


def kernel(col_hidden, row_hidden, obj_hidden, cv_supp, vc_supp, vo_supp,
           co_supp, vc_w, cv_w, co_w, oc_w, vo_w, ov_w,
           attn_vo_w, attn_vo_b, attn_cv_w, attn_cv_b,
           attn_co_w, attn_co_b, attn_vc_w, attn_vc_b):
    row_next, cv_copy, obj1 = _flash_conv(
        cv_supp[0], col_hidden, row_hidden, vo_supp, obj_hidden,
        attn_vo_w[:H], attn_vo_w[H:H + 1], attn_vo_w[H + 1:],
        attn_vo_b.reshape(1, 1), vo_w,
        attn_cv_w[:H], attn_cv_w[H + 1:], attn_cv_w[H:H + 1],
        attn_cv_b.reshape(1, 1), oc_w, vc_w)
    col_next, vc_copy, obj2 = _flash_conv(
        vc_supp[0], row_next, col_hidden, co_supp, obj1,
        attn_co_w[:H], attn_co_w[H:H + 1], attn_co_w[H + 1:],
        attn_co_b.reshape(1, 1), co_w,
        attn_vc_w[:H], attn_vc_w[H + 1:], attn_vc_w[H:H + 1],
        attn_vc_b.reshape(1, 1), ov_w, cv_w)
    return (col_next, row_next, obj2, cv_copy[None], vc_copy[None],
            vo_supp, co_supp)
